# baseline (speedup 1.0000x reference)
import numpy as np
import jax
import jax.numpy as jnp
from jax.experimental import pallas as pl
from jax.experimental.pallas import tpu as pltpu

ALPHA = 4


def _make_body(idx):
    idx = [int(t) for t in idx]

    def body(in_hbm, slow_hbm, fast_hbm, sem_fast, sem_slow):
        fast_cp = pltpu.make_async_copy(in_hbm, fast_hbm, sem_fast)
        fast_cp.start()
        cps = []
        for k, t in enumerate(idx):
            cp = pltpu.make_async_copy(
                in_hbm.at[:, pl.ds(t, 1)], slow_hbm.at[:, pl.ds(k, 1)], sem_slow
            )
            cp.start()
            cps.append(cp)
        for cp in cps:
            cp.wait()
        fast_cp.wait()

    return body


def kernel(frames):
    C, T, H, W = frames.shape
    N = T // ALPHA

    idx = np.linspace(0, T - 1, N).astype(np.int64)

    slow, fast = pl.pallas_call(
        _make_body(idx),
        in_specs=[pl.BlockSpec(memory_space=pltpu.MemorySpace.HBM)],
        out_specs=(
            pl.BlockSpec(memory_space=pltpu.MemorySpace.HBM),
            pl.BlockSpec(memory_space=pltpu.MemorySpace.HBM),
        ),
        out_shape=(
            jax.ShapeDtypeStruct((C, N, H, W), frames.dtype),
            jax.ShapeDtypeStruct((C, T, H, W), frames.dtype),
        ),
        scratch_shapes=[pltpu.SemaphoreType.DMA, pltpu.SemaphoreType.DMA],
    )(frames)
    return (slow, fast)


# speedup vs baseline: 51.1618x; 51.1618x over previous
import numpy as np
import jax
import jax.numpy as jnp
from jax.experimental import pallas as pl
from jax.experimental.pallas import tpu as pltpu

ALPHA = 4


def _pack_body(in_ref, slow_hbm, fast_hbm, sem_fast, sem_slow, *, a, b):
    s = pl.program_id(0)
    fast_cp = pltpu.make_async_copy(
        in_ref, fast_hbm.at[:, pl.ds(s * ALPHA, ALPHA)], sem_fast
    )
    fast_cp.start()
    loc = (s * a) // b - ALPHA * s
    slow_cp = pltpu.make_async_copy(
        in_ref.at[:, pl.ds(loc, 1)], slow_hbm.at[:, pl.ds(s, 1)], sem_slow
    )
    slow_cp.start()
    slow_cp.wait()
    fast_cp.wait()


def kernel(frames):
    C, T, H, W = frames.shape
    N = T // ALPHA
    a, b = T - 1, N - 1

    idx = np.linspace(0, T - 1, N).astype(np.int64)
    idx_arith = (np.arange(N) * a) // b
    assert np.array_equal(idx, idx_arith), (idx, idx_arith)
    assert np.array_equal(idx // ALPHA, np.arange(N)), idx

    slow, fast = pl.pallas_call(
        lambda i, so, fo, s1, s2: _pack_body(i, so, fo, s1, s2, a=a, b=b),
        grid=(N,),
        in_specs=[pl.BlockSpec((C, ALPHA, H, W), lambda s: (0, s, 0, 0))],
        out_specs=(
            pl.BlockSpec(memory_space=pltpu.MemorySpace.HBM),
            pl.BlockSpec(memory_space=pltpu.MemorySpace.HBM),
        ),
        out_shape=(
            jax.ShapeDtypeStruct((C, N, H, W), frames.dtype),
            jax.ShapeDtypeStruct((C, T, H, W), frames.dtype),
        ),
        scratch_shapes=[pltpu.SemaphoreType.DMA, pltpu.SemaphoreType.DMA],
    )(frames)
    return (slow, fast)


# speedup vs baseline: 53.4151x; 1.0440x over previous
import numpy as np
import jax
import jax.numpy as jnp
from jax.experimental import pallas as pl

ALPHA = 4
FB = 8


def _pack_body(in_ref, slow_ref, fast_ref, *, a, b):
    s = pl.program_id(0)
    fast_ref[...] = in_ref[...]
    loc0 = (2 * s * a) // b - FB * s
    loc1 = ((2 * s + 1) * a) // b - FB * s
    slow_ref[:, pl.ds(0, 1)] = in_ref[:, pl.ds(loc0, 1)]
    slow_ref[:, pl.ds(1, 1)] = in_ref[:, pl.ds(loc1, 1)]


def kernel(frames):
    C, T, H, W = frames.shape
    N = T // ALPHA
    a, b = T - 1, N - 1

    idx = np.linspace(0, T - 1, N).astype(np.int64)
    idx_arith = (np.arange(N) * a) // b
    assert np.array_equal(idx, idx_arith), (idx, idx_arith)
    assert np.array_equal(idx // FB, np.arange(N) // 2), idx

    slow, fast = pl.pallas_call(
        lambda i, s, f: _pack_body(i, s, f, a=a, b=b),
        grid=(T // FB, 2),
        in_specs=[
            pl.BlockSpec((C, FB, H // 2, W), lambda s, h: (0, s, h, 0))
        ],
        out_specs=(
            pl.BlockSpec((C, 2, H // 2, W), lambda s, h: (0, s, h, 0)),
            pl.BlockSpec((C, FB, H // 2, W), lambda s, h: (0, s, h, 0)),
        ),
        out_shape=(
            jax.ShapeDtypeStruct((C, N, H, W), frames.dtype),
            jax.ShapeDtypeStruct((C, T, H, W), frames.dtype),
        ),
    )(frames)
    return (slow, fast)
